# baseline (speedup 1.0000x reference)
import functools

import jax
import jax.numpy as jnp
from jax import lax
from jax.experimental import pallas as pl
from jax.experimental.pallas import tpu as pltpu
from jax.experimental.pallas import tpu_sc as plsc

N_NODES = 10000
N_EDGES = 320000
D = 128
DH = D // 2

NC = 2
NS = 16

CHUNK = 128
EROWS = 2560
E_PAD = EROWS * CHUNK - N_EDGES
ROWS_PER_TILE = EROWS // NS
DEG_SPLIT = ROWS_PER_TILE // 2
N_ACC = N_NODES + 8
NODES_PER_TILE = N_NODES // NS
ZCH = 125
WB_ROWS = NODES_PER_TILE // 8 * 8
DEG_W = 16


def _prime(nbuf, x_hbm, src_v, bufs, gsems):
    for b in range(nbuf):
        pltpu.async_copy(x_hbm.at[src_v.at[b]], bufs[b], gsems[b])


def _edge_loop(nbuf, x_hbm, src_v, dst_v, bufs, ones_v, agg_sh, deg_sh,
               gsems, ssems, deg_lo, deg_hi):
    @pl.loop(0, ROWS_PER_TILE, step=nbuf)
    def _(c):
        for b in range(nbuf):
            k = c + b
            pltpu.make_async_copy(x_hbm.at[src_v.at[k]],
                                  bufs[b], gsems[b]).wait()
            sdesc = pltpu.async_copy(bufs[b], agg_sh.at[dst_v.at[k]],
                                     ssems[b], add=True)

            if deg_sh is not None:
                @pl.when((k >= deg_lo) & (k < deg_hi))
                def _():
                    pltpu.sync_copy(ones_v, deg_sh.at[dst_v.at[k]], add=True)

            sdesc.wait()

            @pl.when(k + nbuf < ROWS_PER_TILE)
            def _():
                pltpu.async_copy(x_hbm.at[src_v.at[k + nbuf]],
                                 bufs[b], gsems[b])


def _sc_agg_body(with_deg, nbuf, *refs):
    if with_deg:
        (e_hbm, x0_hbm, x1_hbm, out_hbm, deg_hbm, src_v, dst_v,
         b0, b1, zero_v, ones_v, zdeg_v, agg_sh, deg_sh,
         g0, g1, s0, s1) = refs
        bufs, gsems, ssems = (b0, b1), (g0, g1), (s0, s1)
    else:
        (e_hbm, x0_hbm, x1_hbm, out_hbm, src_v, dst_v,
         b0, b1, b2, b3, zero_v, agg_sh,
         g0, g1, g2, g3, s0, s1, s2, s3) = refs
        bufs, gsems, ssems = (b0, b1, b2, b3), (g0, g1, g2, g3), (s0, s1, s2, s3)
        ones_v = zdeg_v = deg_sh = deg_hbm = None

    cid = lax.axis_index("c")
    sid = lax.axis_index("s")

    zf32 = jnp.zeros((16,), jnp.float32)
    of32 = jnp.full((16,), 1.0, jnp.float32)

    @pl.loop(0, ZCH)
    def _(i):
        for j in range(DH // 16):
            zero_v[i, pl.ds(16 * j, 16)] = zf32
        if with_deg:
            ones_v[i, :] = of32
            zdeg_v[i, :] = zf32

    if with_deg:
        @pl.loop(ZCH, CHUNK)
        def _(i):
            ones_v[i, :] = of32

    node_base = sid * NODES_PER_TILE
    for k in range(NODES_PER_TILE // ZCH):
        pltpu.sync_copy(zero_v, agg_sh.at[pl.ds(node_base + k * ZCH, ZCH)])
        if with_deg:
            pltpu.sync_copy(zdeg_v, deg_sh.at[pl.ds(node_base + k * ZCH, ZCH)])

    @pl.when(sid == 0)
    def _():
        pltpu.sync_copy(zero_v.at[pl.ds(0, N_ACC - N_NODES)],
                        agg_sh.at[pl.ds(N_NODES, N_ACC - N_NODES)])
        if with_deg:
            pltpu.sync_copy(zdeg_v.at[pl.ds(0, N_ACC - N_NODES)],
                            deg_sh.at[pl.ds(N_NODES, N_ACC - N_NODES)])

    row_base = sid * ROWS_PER_TILE
    pltpu.sync_copy(e_hbm.at[0, pl.ds(row_base, ROWS_PER_TILE)], src_v)
    pltpu.sync_copy(e_hbm.at[1, pl.ds(row_base, ROWS_PER_TILE)], dst_v)

    @pl.when(cid == 0)
    def _():
        _prime(nbuf, x0_hbm, src_v, bufs, gsems)

    @pl.when(cid == 1)
    def _():
        _prime(nbuf, x1_hbm, src_v, bufs, gsems)

    plsc.subcore_barrier()

    @pl.when(cid == 0)
    def _():
        _edge_loop(nbuf, x0_hbm, src_v, dst_v, bufs, ones_v, agg_sh, deg_sh,
                   gsems, ssems, 0, DEG_SPLIT)

    @pl.when(cid == 1)
    def _():
        _edge_loop(nbuf, x1_hbm, src_v, dst_v, bufs, ones_v, agg_sh, deg_sh,
                   gsems, ssems, DEG_SPLIT, ROWS_PER_TILE)

    plsc.subcore_barrier()

    wb_base = sid * WB_ROWS

    @pl.when(cid == 0)
    def _():
        pltpu.sync_copy(agg_sh.at[pl.ds(wb_base, WB_ROWS)],
                        out_hbm.at[pl.ds(wb_base, WB_ROWS), pl.ds(0, DH)])

    @pl.when(cid == 1)
    def _():
        pltpu.sync_copy(agg_sh.at[pl.ds(wb_base, WB_ROWS)],
                        out_hbm.at[pl.ds(wb_base, WB_ROWS), pl.ds(DH, DH)])

    if with_deg:
        pltpu.sync_copy(deg_sh.at[pl.ds(wb_base, WB_ROWS)],
                        deg_hbm.at[cid, pl.ds(wb_base, WB_ROWS)])

    @pl.when(sid == 0)
    def _():
        tail = N_NODES - NS * WB_ROWS

        @pl.when(cid == 0)
        def _():
            pltpu.sync_copy(agg_sh.at[pl.ds(NS * WB_ROWS, tail)],
                            out_hbm.at[pl.ds(NS * WB_ROWS, tail), pl.ds(0, DH)])

        @pl.when(cid == 1)
        def _():
            pltpu.sync_copy(agg_sh.at[pl.ds(NS * WB_ROWS, tail)],
                            out_hbm.at[pl.ds(NS * WB_ROWS, tail), pl.ds(DH, DH)])

        if with_deg:
            pltpu.sync_copy(deg_sh.at[pl.ds(NS * WB_ROWS, tail)],
                            deg_hbm.at[cid, pl.ds(NS * WB_ROWS, tail)])


def _make_sc(with_deg, nbuf):
    out_type = [jax.ShapeDtypeStruct((N_NODES, D), jnp.float32)]
    scratch = [
        pltpu.VMEM((ROWS_PER_TILE, CHUNK), jnp.int32),
        pltpu.VMEM((ROWS_PER_TILE, CHUNK), jnp.int32),
    ]
    scratch += [pltpu.VMEM((CHUNK, DH), jnp.float32)] * nbuf
    scratch.append(pltpu.VMEM((ZCH, DH), jnp.float32))
    if with_deg:
        out_type.append(jax.ShapeDtypeStruct((NC, N_NODES, DEG_W), jnp.float32))
        scratch += [
            pltpu.VMEM((CHUNK, DEG_W), jnp.float32),
            pltpu.VMEM((ZCH, DEG_W), jnp.float32),
        ]
    scratch.append(pltpu.VMEM_SHARED((N_ACC, DH), jnp.float32))
    if with_deg:
        scratch.append(pltpu.VMEM_SHARED((N_ACC, DEG_W), jnp.float32))
    scratch += [pltpu.SemaphoreType.DMA] * (2 * nbuf)

    return pl.kernel(
        functools.partial(_sc_agg_body, with_deg, nbuf),
        out_type=tuple(out_type) if with_deg else out_type[0],
        mesh=plsc.VectorSubcoreMesh(core_axis_name="c", subcore_axis_name="s",
                                    num_cores=NC, num_subcores=NS),
        scratch_types=tuple(scratch),
        compiler_params=pltpu.CompilerParams(use_tc_tiling_on_sc=False),
        name="sc_sage_agg_deg" if with_deg else "sc_sage_agg",
    )


_sc_agg_deg = _make_sc(True, 2)
_sc_agg2 = _make_sc(False, 4)


def _tc_body(relu, p_ref, dp_ref, x_ref, wl_ref, b_ref, wr_ref, o_ref):
    deg = dp_ref[0, :, 0:1] + dp_ref[1, :, 0:1]
    a = p_ref[...] / jnp.maximum(deg, 1.0)
    o = (jnp.dot(a, wl_ref[...], preferred_element_type=jnp.float32)
         + b_ref[...]
         + jnp.dot(x_ref[...], wr_ref[...], preferred_element_type=jnp.float32))
    o_ref[...] = jnp.maximum(o, 0.0) if relu else o


def _make_tc(relu):
    return pl.pallas_call(
        functools.partial(_tc_body, relu),
        out_shape=jax.ShapeDtypeStruct((N_NODES, D), jnp.float32),
    )


_tc1 = _make_tc(True)
_tc2 = _make_tc(False)


@jax.jit
def kernel(x, edge_index, W1l, b1, W1r, W2l, b2, W2r):
    pad = jnp.concatenate(
        [jnp.zeros((1, E_PAD), jnp.int32),
         jnp.full((1, E_PAD), N_NODES, jnp.int32)], axis=0)
    e4 = jnp.concatenate([edge_index, pad], axis=1).reshape(2, EROWS, CHUNK)

    agg1, degp = _sc_agg_deg(e4, x[:, :DH], x[:, DH:])
    h = _tc1(agg1, degp, x, W1l.T, b1.reshape(1, D), W1r.T)
    agg2 = _sc_agg2(e4, h[:, :DH], h[:, DH:])
    out = _tc2(agg2, degp, h, W2l.T, b2.reshape(1, D), W2r.T)
    return out


# speedup vs baseline: 1.0907x; 1.0907x over previous
import functools

import jax
import jax.numpy as jnp
from jax import lax
from jax.experimental import pallas as pl
from jax.experimental.pallas import tpu as pltpu
from jax.experimental.pallas import tpu_sc as plsc

N_NODES = 10000
N_EDGES = 320000
D = 128
DH = D // 2

NC = 2
NS = 16

CHUNK = 128
EROWS = 2560
E_PAD = EROWS * CHUNK - N_EDGES
ROWS_PER_TILE = EROWS // NS
DEG_SPLIT = ROWS_PER_TILE // 2
N_ACC = N_NODES + 8
NODES_PER_TILE = N_NODES // NS
ZCH = 125
WB_ROWS = NODES_PER_TILE // 8 * 8
DEG_W = 16


def _prime(nbuf, x_hbm, src_v, bufs, gsems):
    for b in range(nbuf):
        pltpu.async_copy(x_hbm.at[src_v.at[b]], bufs[b], gsems[b])


def _edge_loop(nbuf, x_hbm, src_v, dst_v, bufs, ones_v, agg_sh, deg_sh,
               gsems, ssems, deg_lo, deg_hi):
    @pl.loop(0, ROWS_PER_TILE, step=nbuf)
    def _(c):
        for b in range(nbuf):
            k = c + b
            pltpu.make_async_copy(x_hbm.at[src_v.at[k]],
                                  bufs[b], gsems[b]).wait()
            sdesc = pltpu.async_copy(bufs[b], agg_sh.at[dst_v.at[k]],
                                     ssems[b], add=True)

            if deg_sh is not None:
                @pl.when((k >= deg_lo) & (k < deg_hi))
                def _():
                    pltpu.sync_copy(ones_v, deg_sh.at[dst_v.at[k]], add=True)

            sdesc.wait()

            @pl.when(k + nbuf < ROWS_PER_TILE)
            def _():
                pltpu.async_copy(x_hbm.at[src_v.at[k + nbuf]],
                                 bufs[b], gsems[b])


def _sc_agg_body(with_deg, nbuf, *refs):
    if with_deg:
        (e_hbm, x0_hbm, x1_hbm, out_hbm, deg_hbm, src_v, dst_v,
         b0, b1, zero_v, ones_v, zdeg_v, agg_sh, deg_sh,
         g0, g1, s0, s1) = refs
        bufs, gsems, ssems = (b0, b1), (g0, g1), (s0, s1)
    else:
        (e_hbm, x0_hbm, x1_hbm, out_hbm, src_v, dst_v,
         b0, b1, b2, b3, zero_v, agg_sh,
         g0, g1, g2, g3, s0, s1, s2, s3) = refs
        bufs, gsems, ssems = (b0, b1, b2, b3), (g0, g1, g2, g3), (s0, s1, s2, s3)
        ones_v = zdeg_v = deg_sh = deg_hbm = None

    cid = lax.axis_index("c")
    sid = lax.axis_index("s")

    zf32 = jnp.zeros((16,), jnp.float32)
    of32 = jnp.full((16,), 1.0, jnp.float32)

    @pl.loop(0, ZCH)
    def _(i):
        for j in range(DH // 16):
            zero_v[i, pl.ds(16 * j, 16)] = zf32
        if with_deg:
            ones_v[i, :] = of32
            zdeg_v[i, :] = zf32

    if with_deg:
        @pl.loop(ZCH, CHUNK)
        def _(i):
            ones_v[i, :] = of32

    node_base = sid * NODES_PER_TILE
    for k in range(NODES_PER_TILE // ZCH):
        pltpu.sync_copy(zero_v, agg_sh.at[pl.ds(node_base + k * ZCH, ZCH)])
        if with_deg:
            pltpu.sync_copy(zdeg_v, deg_sh.at[pl.ds(node_base + k * ZCH, ZCH)])

    @pl.when(sid == 0)
    def _():
        pltpu.sync_copy(zero_v.at[pl.ds(0, N_ACC - N_NODES)],
                        agg_sh.at[pl.ds(N_NODES, N_ACC - N_NODES)])
        if with_deg:
            pltpu.sync_copy(zdeg_v.at[pl.ds(0, N_ACC - N_NODES)],
                            deg_sh.at[pl.ds(N_NODES, N_ACC - N_NODES)])

    row_base = sid * ROWS_PER_TILE
    pltpu.sync_copy(e_hbm.at[0, pl.ds(row_base, ROWS_PER_TILE)], src_v)
    pltpu.sync_copy(e_hbm.at[1, pl.ds(row_base, ROWS_PER_TILE)], dst_v)

    @pl.when(cid == 0)
    def _():
        _prime(nbuf, x0_hbm, src_v, bufs, gsems)

    @pl.when(cid == 1)
    def _():
        _prime(nbuf, x1_hbm, src_v, bufs, gsems)

    plsc.subcore_barrier()

    @pl.when(cid == 0)
    def _():
        _edge_loop(nbuf, x0_hbm, src_v, dst_v, bufs, ones_v, agg_sh, deg_sh,
                   gsems, ssems, 0, DEG_SPLIT)

    @pl.when(cid == 1)
    def _():
        _edge_loop(nbuf, x1_hbm, src_v, dst_v, bufs, ones_v, agg_sh, deg_sh,
                   gsems, ssems, DEG_SPLIT, ROWS_PER_TILE)

    plsc.subcore_barrier()

    wb_base = sid * WB_ROWS
    pltpu.sync_copy(agg_sh.at[pl.ds(wb_base, WB_ROWS)],
                    out_hbm.at[cid, pl.ds(wb_base, WB_ROWS)])
    if with_deg:
        pltpu.sync_copy(deg_sh.at[pl.ds(wb_base, WB_ROWS)],
                        deg_hbm.at[cid, pl.ds(wb_base, WB_ROWS)])

    @pl.when(sid == 0)
    def _():
        tail = N_NODES - NS * WB_ROWS
        pltpu.sync_copy(agg_sh.at[pl.ds(NS * WB_ROWS, tail)],
                        out_hbm.at[cid, pl.ds(NS * WB_ROWS, tail)])
        if with_deg:
            pltpu.sync_copy(deg_sh.at[pl.ds(NS * WB_ROWS, tail)],
                            deg_hbm.at[cid, pl.ds(NS * WB_ROWS, tail)])


def _make_sc(with_deg, nbuf):
    out_type = [jax.ShapeDtypeStruct((NC, N_NODES, DH), jnp.float32)]
    scratch = [
        pltpu.VMEM((ROWS_PER_TILE, CHUNK), jnp.int32),
        pltpu.VMEM((ROWS_PER_TILE, CHUNK), jnp.int32),
    ]
    scratch += [pltpu.VMEM((CHUNK, DH), jnp.float32)] * nbuf
    scratch.append(pltpu.VMEM((ZCH, DH), jnp.float32))
    if with_deg:
        out_type.append(jax.ShapeDtypeStruct((NC, N_NODES, DEG_W), jnp.float32))
        scratch += [
            pltpu.VMEM((CHUNK, DEG_W), jnp.float32),
            pltpu.VMEM((ZCH, DEG_W), jnp.float32),
        ]
    scratch.append(pltpu.VMEM_SHARED((N_ACC, DH), jnp.float32))
    if with_deg:
        scratch.append(pltpu.VMEM_SHARED((N_ACC, DEG_W), jnp.float32))
    scratch += [pltpu.SemaphoreType.DMA] * (2 * nbuf)

    return pl.kernel(
        functools.partial(_sc_agg_body, with_deg, nbuf),
        out_type=tuple(out_type) if with_deg else out_type[0],
        mesh=plsc.VectorSubcoreMesh(core_axis_name="c", subcore_axis_name="s",
                                    num_cores=NC, num_subcores=NS),
        scratch_types=tuple(scratch),
        compiler_params=pltpu.CompilerParams(use_tc_tiling_on_sc=False),
        name="sc_sage_agg_deg" if with_deg else "sc_sage_agg",
    )


_sc_agg_deg = _make_sc(True, 2)
_sc_agg2 = _make_sc(False, 4)


def _tc_body(relu, p_ref, dp_ref, x_ref, wl_ref, b_ref, wr_ref, o_ref):
    deg = dp_ref[0, :, 0:1] + dp_ref[1, :, 0:1]
    agg = jnp.concatenate((p_ref[0], p_ref[1]), axis=1)
    a = agg / jnp.maximum(deg, 1.0)
    o = (jnp.dot(a, wl_ref[...], preferred_element_type=jnp.float32)
         + b_ref[...]
         + jnp.dot(x_ref[...], wr_ref[...], preferred_element_type=jnp.float32))
    o_ref[...] = jnp.maximum(o, 0.0) if relu else o


def _make_tc(relu):
    return pl.pallas_call(
        functools.partial(_tc_body, relu),
        out_shape=jax.ShapeDtypeStruct((N_NODES, D), jnp.float32),
    )


_tc1 = _make_tc(True)
_tc2 = _make_tc(False)


@jax.jit
def kernel(x, edge_index, W1l, b1, W1r, W2l, b2, W2r):
    pad = jnp.concatenate(
        [jnp.zeros((1, E_PAD), jnp.int32),
         jnp.full((1, E_PAD), N_NODES, jnp.int32)], axis=0)
    e4 = jnp.concatenate([edge_index, pad], axis=1).reshape(2, EROWS, CHUNK)

    agg1, degp = _sc_agg_deg(e4, x[:, :DH], x[:, DH:])
    h = _tc1(agg1, degp, x, W1l.T, b1.reshape(1, D), W1r.T)
    agg2 = _sc_agg2(e4, h[:, :DH], h[:, DH:])
    out = _tc2(agg2, degp, h, W2l.T, b2.reshape(1, D), W2r.T)
    return out


# speedup vs baseline: 2.2892x; 2.0989x over previous
import functools

import jax
import jax.numpy as jnp
from jax import lax
from jax.experimental import pallas as pl
from jax.experimental.pallas import tpu as pltpu
from jax.experimental.pallas import tpu_sc as plsc

N_NODES = 10000
N_EDGES = 320000
D = 128
DH = D // 2

NC = 2
NS = 16

CHUNK = 125
EROWS = N_EDGES // CHUNK
ROWS_PER_TILE = EROWS // NS
DEG_SPLIT = ROWS_PER_TILE // 2
NODES_PER_TILE = N_NODES // NS
ZCH = 125
WB_ROWS = NODES_PER_TILE // 8 * 8
DEG_W = 16


def _prime(nbuf, x_hbm, src_v, bufs, gsems):
    for b in range(nbuf):
        pltpu.async_copy(x_hbm.at[src_v.at[b]], bufs[b], gsems[b])


def _edge_loop(nbuf, x_hbm, src_v, dst_v, bufs, ones_v, agg_sh, deg_sh,
               gsems, ssems, deg_lo, deg_hi):
    @pl.loop(0, ROWS_PER_TILE, step=nbuf)
    def _(c):
        for b in range(nbuf):
            k = c + b
            pltpu.make_async_copy(x_hbm.at[src_v.at[k]],
                                  bufs[b], gsems[b]).wait()
            sdesc = pltpu.async_copy(bufs[b], agg_sh.at[dst_v.at[k]],
                                     ssems[b], add=True)

            if deg_sh is not None:
                @pl.when((k >= deg_lo) & (k < deg_hi))
                def _():
                    pltpu.sync_copy(ones_v, deg_sh.at[dst_v.at[k]], add=True)

            sdesc.wait()

            @pl.when(k + nbuf < ROWS_PER_TILE)
            def _():
                pltpu.async_copy(x_hbm.at[src_v.at[k + nbuf]],
                                 bufs[b], gsems[b])


def _sc_agg_body(with_deg, nbuf, *refs):
    if with_deg:
        (e_hbm, x0_hbm, x1_hbm, out_hbm, deg_hbm, src_v, dst_v,
         b0, b1, zero_v, ones_v, zdeg_v, agg_sh, deg_sh,
         g0, g1, s0, s1) = refs
        bufs, gsems, ssems = (b0, b1), (g0, g1), (s0, s1)
    else:
        (e_hbm, x0_hbm, x1_hbm, out_hbm, src_v, dst_v,
         b0, b1, b2, b3, zero_v, agg_sh,
         g0, g1, g2, g3, s0, s1, s2, s3) = refs
        bufs, gsems, ssems = (b0, b1, b2, b3), (g0, g1, g2, g3), (s0, s1, s2, s3)
        ones_v = zdeg_v = deg_sh = deg_hbm = None

    cid = lax.axis_index("c")
    sid = lax.axis_index("s")

    zf32 = jnp.zeros((16,), jnp.float32)
    of32 = jnp.full((16,), 1.0, jnp.float32)

    @pl.loop(0, ZCH)
    def _(i):
        for j in range(DH // 16):
            zero_v[i, pl.ds(16 * j, 16)] = zf32
        if with_deg:
            ones_v[i, :] = of32
            zdeg_v[i, :] = zf32

    node_base = sid * NODES_PER_TILE
    for k in range(NODES_PER_TILE // ZCH):
        pltpu.sync_copy(zero_v, agg_sh.at[pl.ds(node_base + k * ZCH, ZCH)])
        if with_deg:
            pltpu.sync_copy(zdeg_v, deg_sh.at[pl.ds(node_base + k * ZCH, ZCH)])

    row_base = sid * ROWS_PER_TILE
    pltpu.sync_copy(e_hbm.at[0, pl.ds(row_base, ROWS_PER_TILE)], src_v)
    pltpu.sync_copy(e_hbm.at[1, pl.ds(row_base, ROWS_PER_TILE)], dst_v)

    @pl.when(cid == 0)
    def _():
        _prime(nbuf, x0_hbm, src_v, bufs, gsems)

    @pl.when(cid == 1)
    def _():
        _prime(nbuf, x1_hbm, src_v, bufs, gsems)

    plsc.subcore_barrier()

    @pl.when(cid == 0)
    def _():
        _edge_loop(nbuf, x0_hbm, src_v, dst_v, bufs, ones_v, agg_sh, deg_sh,
                   gsems, ssems, 0, DEG_SPLIT)

    @pl.when(cid == 1)
    def _():
        _edge_loop(nbuf, x1_hbm, src_v, dst_v, bufs, ones_v, agg_sh, deg_sh,
                   gsems, ssems, DEG_SPLIT, ROWS_PER_TILE)

    plsc.subcore_barrier()

    wb_base = sid * WB_ROWS
    pltpu.sync_copy(agg_sh.at[pl.ds(wb_base, WB_ROWS)],
                    out_hbm.at[cid, pl.ds(wb_base, WB_ROWS)])
    if with_deg:
        pltpu.sync_copy(deg_sh.at[pl.ds(wb_base, WB_ROWS)],
                        deg_hbm.at[cid, pl.ds(wb_base, WB_ROWS)])

    @pl.when(sid == 0)
    def _():
        tail = N_NODES - NS * WB_ROWS
        pltpu.sync_copy(agg_sh.at[pl.ds(NS * WB_ROWS, tail)],
                        out_hbm.at[cid, pl.ds(NS * WB_ROWS, tail)])
        if with_deg:
            pltpu.sync_copy(deg_sh.at[pl.ds(NS * WB_ROWS, tail)],
                            deg_hbm.at[cid, pl.ds(NS * WB_ROWS, tail)])


def _make_sc(with_deg, nbuf):
    out_type = [jax.ShapeDtypeStruct((NC, N_NODES, DH), jnp.float32)]
    scratch = [
        pltpu.VMEM((ROWS_PER_TILE, CHUNK), jnp.int32),
        pltpu.VMEM((ROWS_PER_TILE, CHUNK), jnp.int32),
    ]
    scratch += [pltpu.VMEM((CHUNK, DH), jnp.float32)] * nbuf
    scratch.append(pltpu.VMEM((ZCH, DH), jnp.float32))
    if with_deg:
        out_type.append(jax.ShapeDtypeStruct((NC, N_NODES, DEG_W), jnp.float32))
        scratch += [
            pltpu.VMEM((CHUNK, DEG_W), jnp.float32),
            pltpu.VMEM((ZCH, DEG_W), jnp.float32),
        ]
    scratch.append(pltpu.VMEM_SHARED((N_NODES, DH), jnp.float32))
    if with_deg:
        scratch.append(pltpu.VMEM_SHARED((N_NODES, DEG_W), jnp.float32))
    scratch += [pltpu.SemaphoreType.DMA] * (2 * nbuf)

    return pl.kernel(
        functools.partial(_sc_agg_body, with_deg, nbuf),
        out_type=tuple(out_type) if with_deg else out_type[0],
        mesh=plsc.VectorSubcoreMesh(core_axis_name="c", subcore_axis_name="s",
                                    num_cores=NC, num_subcores=NS),
        scratch_types=tuple(scratch),
        compiler_params=pltpu.CompilerParams(use_tc_tiling_on_sc=False),
        name="sc_sage_agg_deg" if with_deg else "sc_sage_agg",
    )


_sc_agg_deg = _make_sc(True, 2)
_sc_agg2 = _make_sc(False, 4)


def _tc_body(relu, p_ref, dp_ref, x_ref, wl_ref, b_ref, wr_ref, o_ref):
    deg = dp_ref[0, :, 0:1] + dp_ref[1, :, 0:1]
    agg = jnp.concatenate((p_ref[0], p_ref[1]), axis=1)
    a = agg / jnp.maximum(deg, 1.0)
    o = (jnp.dot(a, wl_ref[...], preferred_element_type=jnp.float32)
         + b_ref[...]
         + jnp.dot(x_ref[...], wr_ref[...], preferred_element_type=jnp.float32))
    o_ref[...] = jnp.maximum(o, 0.0) if relu else o


def _make_tc(relu):
    return pl.pallas_call(
        functools.partial(_tc_body, relu),
        out_shape=jax.ShapeDtypeStruct((N_NODES, D), jnp.float32),
    )


_tc1 = _make_tc(True)
_tc2 = _make_tc(False)


@jax.jit
def kernel(x, edge_index, W1l, b1, W1r, W2l, b2, W2r):
    e4 = edge_index.reshape(2, EROWS, CHUNK)

    agg1, degp = _sc_agg_deg(e4, x[:, :DH], x[:, DH:])
    h = _tc1(agg1, degp, x, W1l.T, b1.reshape(1, D), W1r.T)
    agg2 = _sc_agg2(e4, h[:, :DH], h[:, DH:])
    out = _tc2(agg2, degp, h, W2l.T, b2.reshape(1, D), W2r.T)
    return out


# speedup vs baseline: 2.4428x; 1.0671x over previous
import functools

import jax
import jax.numpy as jnp
from jax import lax
from jax.experimental import pallas as pl
from jax.experimental.pallas import tpu as pltpu
from jax.experimental.pallas import tpu_sc as plsc

N_NODES = 10000
N_EDGES = 320000
D = 128
DH = D // 2

NC = 2
NS = 16

CHUNK = 125
EROWS = N_EDGES // CHUNK
ROWS_PER_TILE = EROWS // NS
DEG_SPLIT = ROWS_PER_TILE // 2
NODES_PER_TILE = N_NODES // NS
ZCH = 125
WB_ROWS = NODES_PER_TILE // 8 * 8
DEG_W = 16


def _prime(nbuf, x_hbm, src_v, bufs, gsems):
    for b in range(nbuf):
        pltpu.async_copy(x_hbm.at[src_v.at[b]], bufs[b], gsems[b])


def _edge_loop_merged(nbuf, xs_hbm, cid, src_v, dst_v, bufs, ones_v, agg_sh,
                      deg_sh, gsems, ssems, deg_lo, deg_hi):
    x_hbm = xs_hbm.at[cid]
    full = ROWS_PER_TILE - ROWS_PER_TILE % nbuf

    def _slot(k, b):
        pltpu.make_async_copy(x_hbm.at[src_v.at[k]], bufs[b], gsems[b]).wait()
        sdesc = pltpu.async_copy(bufs[b], agg_sh.at[dst_v.at[k]],
                                 ssems[b], add=True)

        if deg_sh is not None:
            @pl.when((k >= deg_lo) & (k < deg_hi))
            def _():
                pltpu.sync_copy(ones_v, deg_sh.at[dst_v.at[k]], add=True)

        sdesc.wait()

        @pl.when(k + nbuf < ROWS_PER_TILE)
        def _():
            pltpu.async_copy(x_hbm.at[src_v.at[k + nbuf]], bufs[b], gsems[b])

    @pl.loop(0, full, step=nbuf)
    def _(c):
        for b in range(nbuf):
            _slot(c + b, b)

    for i in range(ROWS_PER_TILE % nbuf):
        _slot(full + i, i)


def _sc_agg_body(with_deg, nbuf, *refs):
    if with_deg:
        (e_hbm, xs_hbm, out_hbm, deg_hbm, src_v, dst_v,
         b0, b1, b2, b3, zero_v, ones_v, zdeg_v, agg_sh, deg_sh,
         gsem, ssem) = refs
    else:
        (e_hbm, xs_hbm, out_hbm, src_v, dst_v,
         b0, b1, b2, b3, zero_v, agg_sh,
         gsem, ssem) = refs
        ones_v = zdeg_v = deg_sh = deg_hbm = None
    bufs = (b0, b1, b2, b3)[:nbuf]
    gsems = (gsem,) * nbuf
    ssems = (ssem,) * nbuf

    cid = lax.axis_index("c")
    sid = lax.axis_index("s")

    zf32 = jnp.zeros((16,), jnp.float32)
    of32 = jnp.full((16,), 1.0, jnp.float32)

    @pl.loop(0, ZCH)
    def _(i):
        for j in range(DH // 16):
            zero_v[i, pl.ds(16 * j, 16)] = zf32
        if with_deg:
            ones_v[i, :] = of32
            zdeg_v[i, :] = zf32

    node_base = sid * NODES_PER_TILE
    for k in range(NODES_PER_TILE // ZCH):
        pltpu.sync_copy(zero_v, agg_sh.at[pl.ds(node_base + k * ZCH, ZCH)])
        if with_deg:
            pltpu.sync_copy(zdeg_v, deg_sh.at[pl.ds(node_base + k * ZCH, ZCH)])

    row_base = sid * ROWS_PER_TILE
    pltpu.sync_copy(e_hbm.at[0, pl.ds(row_base, ROWS_PER_TILE)], src_v)
    pltpu.sync_copy(e_hbm.at[1, pl.ds(row_base, ROWS_PER_TILE)], dst_v)

    _prime(nbuf, xs_hbm.at[cid], src_v, bufs, gsems)

    plsc.subcore_barrier()

    deg_lo = cid * DEG_SPLIT
    _edge_loop_merged(nbuf, xs_hbm, cid, src_v, dst_v, bufs, ones_v, agg_sh,
                      deg_sh, gsems, ssems, deg_lo, deg_lo + DEG_SPLIT)

    plsc.subcore_barrier()

    wb_base = sid * WB_ROWS
    pltpu.sync_copy(agg_sh.at[pl.ds(wb_base, WB_ROWS)],
                    out_hbm.at[cid, pl.ds(wb_base, WB_ROWS)])
    if with_deg:
        pltpu.sync_copy(deg_sh.at[pl.ds(wb_base, WB_ROWS)],
                        deg_hbm.at[cid, pl.ds(wb_base, WB_ROWS)])

    @pl.when(sid == 0)
    def _():
        tail = N_NODES - NS * WB_ROWS
        pltpu.sync_copy(agg_sh.at[pl.ds(NS * WB_ROWS, tail)],
                        out_hbm.at[cid, pl.ds(NS * WB_ROWS, tail)])
        if with_deg:
            pltpu.sync_copy(deg_sh.at[pl.ds(NS * WB_ROWS, tail)],
                            deg_hbm.at[cid, pl.ds(NS * WB_ROWS, tail)])


def _make_sc(with_deg, nbuf):
    out_type = [jax.ShapeDtypeStruct((NC, N_NODES, DH), jnp.float32)]
    scratch = [
        pltpu.VMEM((ROWS_PER_TILE, CHUNK), jnp.int32),
        pltpu.VMEM((ROWS_PER_TILE, CHUNK), jnp.int32),
    ]
    scratch += [pltpu.VMEM((CHUNK, DH), jnp.float32)] * 4
    scratch.append(pltpu.VMEM((ZCH, DH), jnp.float32))
    if with_deg:
        out_type.append(jax.ShapeDtypeStruct((NC, N_NODES, DEG_W), jnp.float32))
        scratch += [
            pltpu.VMEM((CHUNK, DEG_W), jnp.float32),
            pltpu.VMEM((ZCH, DEG_W), jnp.float32),
        ]
    scratch.append(pltpu.VMEM_SHARED((N_NODES, DH), jnp.float32))
    if with_deg:
        scratch.append(pltpu.VMEM_SHARED((N_NODES, DEG_W), jnp.float32))
    scratch += [pltpu.SemaphoreType.DMA] * 2

    return pl.kernel(
        functools.partial(_sc_agg_body, with_deg, nbuf),
        out_type=tuple(out_type) if with_deg else out_type[0],
        mesh=plsc.VectorSubcoreMesh(core_axis_name="c", subcore_axis_name="s",
                                    num_cores=NC, num_subcores=NS),
        scratch_types=tuple(scratch),
        compiler_params=pltpu.CompilerParams(use_tc_tiling_on_sc=False),
        name="sc_sage_agg_deg" if with_deg else "sc_sage_agg",
    )


_sc_agg_deg = _make_sc(True, 3)
_sc_agg2 = _make_sc(False, 4)


def _tc_body(relu, p_ref, dp_ref, x_ref, wl_ref, b_ref, wr_ref, o_ref):
    deg = dp_ref[0, :, 0:1] + dp_ref[1, :, 0:1]
    agg = jnp.concatenate((p_ref[0], p_ref[1]), axis=1)
    a = agg / jnp.maximum(deg, 1.0)
    o = (jnp.dot(a, wl_ref[...], preferred_element_type=jnp.float32)
         + b_ref[...]
         + jnp.dot(x_ref[...], wr_ref[...], preferred_element_type=jnp.float32))
    o_ref[...] = jnp.maximum(o, 0.0) if relu else o


def _make_tc(relu):
    return pl.pallas_call(
        functools.partial(_tc_body, relu),
        out_shape=jax.ShapeDtypeStruct((N_NODES, D), jnp.float32),
    )


_tc1 = _make_tc(True)
_tc2 = _make_tc(False)


@jax.jit
def kernel(x, edge_index, W1l, b1, W1r, W2l, b2, W2r):
    e4 = edge_index.reshape(2, EROWS, CHUNK)

    xs = jnp.stack([x[:, :DH], x[:, DH:]])
    agg1, degp = _sc_agg_deg(e4, xs)
    h = _tc1(agg1, degp, x, W1l.T, b1.reshape(1, D), W1r.T)
    hs = jnp.stack([h[:, :DH], h[:, DH:]])
    agg2 = _sc_agg2(e4, hs)
    out = _tc2(agg2, degp, h, W2l.T, b2.reshape(1, D), W2r.T)
    return out


# speedup vs baseline: 2.5840x; 1.0578x over previous
import functools

import jax
import jax.numpy as jnp
from jax import lax
from jax.experimental import pallas as pl
from jax.experimental.pallas import tpu as pltpu
from jax.experimental.pallas import tpu_sc as plsc

N_NODES = 10000
N_EDGES = 320000
D = 128
DH = D // 2

NC = 2
NS = 16

CHUNK = 125
EROWS = N_EDGES // CHUNK
ROWS_PER_TILE = EROWS // NS
DEG_SPLIT = ROWS_PER_TILE // 2
NODES_PER_TILE = N_NODES // NS
ZCH = 125
WB_ROWS = NODES_PER_TILE // 8 * 8
DEG_W = 16


def _prime(nbuf, x_hbm, src_v, bufs, gsems):
    for b in range(nbuf):
        pltpu.async_copy(x_hbm.at[src_v.at[b]], bufs[b], gsems[b])


def _edge_loop_merged(nbuf, x_hbm, src_v, dst_v, bufs, ones_v, agg_sh,
                      deg_sh, gsems, ssems, deg_lo, deg_hi):
    full = ROWS_PER_TILE - ROWS_PER_TILE % nbuf

    def _slot(k, b):
        pltpu.make_async_copy(x_hbm.at[src_v.at[k]], bufs[b], gsems[b]).wait()
        sdesc = pltpu.async_copy(bufs[b], agg_sh.at[dst_v.at[k]],
                                 ssems[b], add=True)

        if deg_sh is not None:
            @pl.when((k >= deg_lo) & (k < deg_hi))
            def _():
                pltpu.sync_copy(ones_v, deg_sh.at[dst_v.at[k]], add=True)

        sdesc.wait()

        @pl.when(k + nbuf < ROWS_PER_TILE)
        def _():
            pltpu.async_copy(x_hbm.at[src_v.at[k + nbuf]], bufs[b], gsems[b])

    @pl.loop(0, full, step=nbuf)
    def _(c):
        for b in range(nbuf):
            _slot(c + b, b)

    for i in range(ROWS_PER_TILE % nbuf):
        _slot(full + i, i)


def _sc_agg_body(with_deg, nbuf, *refs):
    if with_deg:
        (e_hbm, xs_hbm, out_hbm, deg_hbm, src_v, dst_v,
         b0, b1, b2, b3, zero_v, ones_v, zdeg_v, agg_sh, deg_sh,
         gsem, ssem) = refs
    else:
        (e_hbm, xs_hbm, out_hbm, src_v, dst_v,
         b0, b1, b2, b3, zero_v, agg_sh,
         gsem, ssem) = refs
        ones_v = zdeg_v = deg_sh = deg_hbm = None
    bufs = (b0, b1, b2, b3)[:nbuf]
    gsems = (gsem,) * nbuf
    ssems = (ssem,) * nbuf

    cid = lax.axis_index("c")
    sid = lax.axis_index("s")

    zf32 = jnp.zeros((16,), jnp.float32)
    of32 = jnp.full((16,), 1.0, jnp.float32)

    @pl.loop(0, ZCH)
    def _(i):
        for j in range(DH // 16):
            zero_v[i, pl.ds(16 * j, 16)] = zf32
        if with_deg:
            ones_v[i, :] = of32
            zdeg_v[i, :] = zf32

    node_base = sid * NODES_PER_TILE
    for k in range(NODES_PER_TILE // ZCH):
        pltpu.sync_copy(zero_v, agg_sh.at[pl.ds(node_base + k * ZCH, ZCH)])
        if with_deg:
            pltpu.sync_copy(zdeg_v, deg_sh.at[pl.ds(node_base + k * ZCH, ZCH)])

    row_base = sid * ROWS_PER_TILE
    pltpu.sync_copy(e_hbm.at[cid, pl.ds(row_base, ROWS_PER_TILE)], src_v)
    pltpu.sync_copy(e_hbm.at[2, pl.ds(row_base, ROWS_PER_TILE)], dst_v)

    _prime(nbuf, xs_hbm, src_v, bufs, gsems)

    plsc.subcore_barrier()

    deg_lo = cid * DEG_SPLIT
    _edge_loop_merged(nbuf, xs_hbm, src_v, dst_v, bufs, ones_v, agg_sh,
                      deg_sh, gsems, ssems, deg_lo, deg_lo + DEG_SPLIT)

    plsc.subcore_barrier()

    wb_base = sid * WB_ROWS
    pltpu.sync_copy(agg_sh.at[pl.ds(wb_base, WB_ROWS)],
                    out_hbm.at[cid, pl.ds(wb_base, WB_ROWS)])
    if with_deg:
        pltpu.sync_copy(deg_sh.at[pl.ds(wb_base, WB_ROWS)],
                        deg_hbm.at[cid, pl.ds(wb_base, WB_ROWS)])

    @pl.when(sid == 0)
    def _():
        tail = N_NODES - NS * WB_ROWS
        pltpu.sync_copy(agg_sh.at[pl.ds(NS * WB_ROWS, tail)],
                        out_hbm.at[cid, pl.ds(NS * WB_ROWS, tail)])
        if with_deg:
            pltpu.sync_copy(deg_sh.at[pl.ds(NS * WB_ROWS, tail)],
                            deg_hbm.at[cid, pl.ds(NS * WB_ROWS, tail)])


def _make_sc(with_deg, nbuf):
    out_type = [jax.ShapeDtypeStruct((NC, N_NODES, DH), jnp.float32)]
    scratch = [
        pltpu.VMEM((ROWS_PER_TILE, CHUNK), jnp.int32),
        pltpu.VMEM((ROWS_PER_TILE, CHUNK), jnp.int32),
    ]
    scratch += [pltpu.VMEM((CHUNK, DH), jnp.float32)] * 4
    scratch.append(pltpu.VMEM((ZCH, DH), jnp.float32))
    if with_deg:
        out_type.append(jax.ShapeDtypeStruct((NC, N_NODES, DEG_W), jnp.float32))
        scratch += [
            pltpu.VMEM((CHUNK, DEG_W), jnp.float32),
            pltpu.VMEM((ZCH, DEG_W), jnp.float32),
        ]
    scratch.append(pltpu.VMEM_SHARED((N_NODES, DH), jnp.float32))
    if with_deg:
        scratch.append(pltpu.VMEM_SHARED((N_NODES, DEG_W), jnp.float32))
    scratch += [pltpu.SemaphoreType.DMA] * 2

    return pl.kernel(
        functools.partial(_sc_agg_body, with_deg, nbuf),
        out_type=tuple(out_type) if with_deg else out_type[0],
        mesh=plsc.VectorSubcoreMesh(core_axis_name="c", subcore_axis_name="s",
                                    num_cores=NC, num_subcores=NS),
        scratch_types=tuple(scratch),
        compiler_params=pltpu.CompilerParams(use_tc_tiling_on_sc=False),
        name="sc_sage_agg_deg" if with_deg else "sc_sage_agg",
    )


_sc_agg_deg = _make_sc(True, 3)
_sc_agg2 = _make_sc(False, 4)


def _tc_body(relu, p_ref, dp_ref, x_ref, wl_ref, b_ref, wr_ref, o_ref):
    deg = dp_ref[0, :, 0:1] + dp_ref[1, :, 0:1]
    agg = jnp.concatenate((p_ref[0], p_ref[1]), axis=1)
    a = agg / jnp.maximum(deg, 1.0)
    o = (jnp.dot(a, wl_ref[...], preferred_element_type=jnp.float32)
         + b_ref[...]
         + jnp.dot(x_ref[...], wr_ref[...], preferred_element_type=jnp.float32))
    o_ref[...] = jnp.maximum(o, 0.0) if relu else o


def _make_tc(relu):
    return pl.pallas_call(
        functools.partial(_tc_body, relu),
        out_shape=jax.ShapeDtypeStruct((N_NODES, D), jnp.float32),
    )


_tc1 = _make_tc(True)
_tc2 = _make_tc(False)


@jax.jit
def kernel(x, edge_index, W1l, b1, W1r, W2l, b2, W2r):
    src2 = edge_index[0] * 2
    e4 = jnp.stack([src2, src2 + 1, edge_index[1]]).reshape(3, EROWS, CHUNK)

    agg1, degp = _sc_agg_deg(e4, x.reshape(2 * N_NODES, DH))
    h = _tc1(agg1, degp, x, W1l.T, b1.reshape(1, D), W1r.T)
    agg2 = _sc_agg2(e4, h.reshape(2 * N_NODES, DH))
    out = _tc2(agg2, degp, h, W2l.T, b2.reshape(1, D), W2r.T)
    return out
